# baseline (speedup 1.0000x reference)
import dataclasses

import jax
import jax.numpy as jnp
from jax import lax
from jax.experimental import pallas as pl
from jax.experimental.pallas import tpu as pltpu
from jax.experimental.pallas import tpu_sc as plsc

N = 10000
E = 320000
D = 128
NC = 2
NS = 16
NW = NC * NS
PER_W = E // NW
CH = 40
NCH = PER_W // CH
ROWS_A = 624
TAIL0 = NS * ROWS_A
TAIL = N - TAIL0


def _sc_segsum(with_counts=False):
    NB = 4 if with_counts else 6
    mesh = plsc.VectorSubcoreMesh(core_axis_name="c", subcore_axis_name="s")
    out_type = [jax.ShapeDtypeStruct((NC, N, D), jnp.float32)]
    scratch = (
        [pltpu.VMEM_SHARED((N, D), jnp.float32)]
        + [pltpu.VMEM((PER_W,), jnp.int32)] * 2
        + [pltpu.VMEM((CH, D), jnp.float32)] * NB
        + [pltpu.SemaphoreType.DMA] * (2 * NB)
    )
    cp = None
    if with_counts:
        out_type.append(jax.ShapeDtypeStruct((NW, N), jnp.float32))
        scratch.append(pltpu.VMEM((N,), jnp.float32))
        cp = pltpu.CompilerParams()
        if "needs_layout_passes" in pltpu.CompilerParams.__dataclass_fields__:
            cp = dataclasses.replace(cp, needs_layout_passes=False)

    def body(x_hbm, src_hbm, dst_hbm, zacc_hbm, *rest):
        if with_counts:
            acc_out, cnt_out, acc_sh, src_v, dst_v = rest[:5]
            bufs_and_sems = rest[5:]
            cnt_v = bufs_and_sems[3 * NB]
        else:
            acc_out, acc_sh, src_v, dst_v = rest[:4]
            bufs_and_sems = rest[4:]
        rows = bufs_and_sems[:NB]
        gsem = bufs_and_sems[NB:2 * NB]
        ssem = bufs_and_sems[2 * NB:3 * NB]
        cid = lax.axis_index("c")
        sid = lax.axis_index("s")
        wid = cid * NS + sid
        row0 = sid * ROWS_A

        pltpu.sync_copy(src_hbm.at[wid], src_v)
        pltpu.sync_copy(dst_hbm.at[wid], dst_v)
        pltpu.sync_copy(zacc_hbm.at[pl.ds(row0, ROWS_A)],
                        acc_sh.at[pl.ds(row0, ROWS_A)])

        @pl.when(sid == NS - 1)
        def _():
            pltpu.sync_copy(zacc_hbm.at[pl.ds(TAIL0, TAIL)],
                            acc_sh.at[pl.ds(TAIL0, TAIL)])

        if with_counts:
            @pl.loop(0, N, step=16)
            def _(j):
                cnt_v[pl.ds(j, 16)] = jnp.zeros((16,), jnp.float32)

        plsc.subcore_barrier()

        def counts(c):
            if with_counts:
                ones16 = jnp.ones((16,), jnp.float32)
                for j in range(CH // 16):
                    plsc.addupdate_scatter(
                        cnt_v, [dst_v[pl.ds(c * CH + j * 16, 16)]], ones16)
                rem = CH % 16
                if rem:
                    mask = lax.iota(jnp.int32, 16) >= (16 - rem)
                    plsc.addupdate_scatter(
                        cnt_v, [dst_v[pl.ds(c * CH + CH - 16, 16)]], ones16,
                        mask=mask)

        def start_gather(c, b):
            pltpu.async_copy(
                x_hbm.at[src_v.at[pl.ds(c * CH, CH)]], rows[b], gsem[b])

        def wait_gather(c, b):
            pltpu.make_async_copy(
                x_hbm.at[src_v.at[pl.ds(c * CH, CH)]], rows[b],
                gsem[b]).wait()

        def start_scatter(c, b):
            pltpu.async_copy(
                rows[b], acc_sh.at[dst_v.at[pl.ds(c * CH, CH)]], ssem[b],
                add=True)

        def wait_scatter(c, b):
            pltpu.make_async_copy(
                rows[b], acc_sh.at[dst_v.at[pl.ds(c * CH, CH)]],
                ssem[b]).wait()

        for c in range(NB - 1):
            start_gather(c, c)

        MAIN = NCH - NCH % NB

        @pl.loop(0, MAIN // NB)
        def _(k):
            for j in range(NB):
                b = j
                c = NB * k + j
                wait_gather(c, b)
                start_scatter(c, b)
                if j == 0:
                    @pl.when(k > 0)
                    def _():
                        wait_scatter(c - 1, (b - 1) % NB)
                else:
                    wait_scatter(c - 1, (b - 1) % NB)

                @pl.when(c + NB - 1 < NCH)
                def _():
                    start_gather(c + NB - 1, (j + NB - 1) % NB)

                counts(c)

        for c in range(MAIN, NCH):
            b = c % NB
            wait_gather(c, b)
            start_scatter(c, b)
            wait_scatter(c - 1, (b - 1) % NB)
            counts(c)
        wait_scatter(NCH - 1, (NCH - 1) % NB)

        plsc.subcore_barrier()

        pltpu.sync_copy(acc_sh.at[pl.ds(row0, ROWS_A)],
                        acc_out.at[cid].at[pl.ds(row0, ROWS_A)])

        @pl.when(sid == NS - 1)
        def _():
            pltpu.sync_copy(acc_sh.at[pl.ds(TAIL0, TAIL)],
                            acc_out.at[cid].at[pl.ds(TAIL0, TAIL)])

        if with_counts:
            pltpu.sync_copy(cnt_v, cnt_out.at[wid])

    return pl.kernel(body, out_type=tuple(out_type), mesh=mesh,
                     scratch_types=scratch, compiler_params=cp)


def _sc_hist():
    mesh = plsc.VectorSubcoreMesh(core_axis_name="c", subcore_axis_name="s")
    out_type = jax.ShapeDtypeStruct((NW, N), jnp.float32)
    scratch = [
        pltpu.VMEM((PER_W,), jnp.int32),
        pltpu.VMEM((N,), jnp.float32),
    ]
    cp = pltpu.CompilerParams()
    if "needs_layout_passes" in pltpu.CompilerParams.__dataclass_fields__:
        cp = dataclasses.replace(cp, needs_layout_passes=False)

    def body(dst_hbm, cnt_out, dst_v, cnt_v):
        cid = lax.axis_index("c")
        sid = lax.axis_index("s")
        wid = cid * NS + sid

        pltpu.sync_copy(dst_hbm.at[wid], dst_v)

        @pl.loop(0, N, step=16)
        def _(j):
            cnt_v[pl.ds(j, 16)] = jnp.zeros((16,), jnp.float32)

        ones16 = jnp.ones((16,), jnp.float32)

        @pl.loop(0, PER_W, step=16)
        def _(j):
            plsc.addupdate_scatter(cnt_v, [dst_v[pl.ds(j, 16)]], ones16)

        pltpu.sync_copy(cnt_v, cnt_out.at[wid])

    return pl.kernel(body, out_type=out_type, mesh=mesh,
                     scratch_types=scratch, compiler_params=cp)


def _tc_layer(relu):
    B = 1000

    def body(acc_ref, cnt_ref, x_ref, wl_ref, b_ref, wr_ref, o_ref):
        s = acc_ref[0] + acc_ref[1]
        cnt = jnp.sum(cnt_ref[...], axis=1)
        mean = s * (1.0 / jnp.maximum(cnt, 1.0))[:, None]
        dn = (((1,), (1,)), ((), ()))
        r = (lax.dot_general(mean, wl_ref[...], dn,
                             preferred_element_type=jnp.float32)
             + lax.dot_general(x_ref[...], wr_ref[...], dn,
                               preferred_element_type=jnp.float32)
             + b_ref[...])
        o_ref[...] = jnp.maximum(r, 0.0) if relu else r

    return pl.pallas_call(
        body,
        grid=(N // B,),
        in_specs=[
            pl.BlockSpec((NC, B, D), lambda i: (0, i, 0)),
            pl.BlockSpec((B, NW), lambda i: (i, 0)),
            pl.BlockSpec((B, D), lambda i: (i, 0)),
            pl.BlockSpec((D, D), lambda i: (0, 0)),
            pl.BlockSpec((1, D), lambda i: (0, 0)),
            pl.BlockSpec((D, D), lambda i: (0, 0)),
        ],
        out_specs=pl.BlockSpec((B, D), lambda i: (i, 0)),
        out_shape=jax.ShapeDtypeStruct((N, D), jnp.float32),
    )


@jax.jit
def kernel(x, edge_index, W1_l, b1_l, W1_r, W2_l, b2_l, W2_r):
    ei = edge_index.astype(jnp.int32)
    src = ei[0].reshape(NW, PER_W)
    dst2 = ei[1].reshape(NW, PER_W)
    zacc = jnp.zeros((N, D), jnp.float32)
    b1 = b1_l.reshape(1, D)
    b2 = b2_l.reshape(1, D)

    acc1, cntp = _sc_segsum(True)(x, src, dst2, zacc)
    cntp = cntp.T
    h = _tc_layer(True)(acc1, cntp, x, W1_l, b1, W1_r)
    (acc2,) = _sc_segsum(False)(h, src, dst2, zacc)
    out = _tc_layer(False)(acc2, cntp, h, W2_l, b2, W2_r)
    return out


# speedup vs baseline: 1.0535x; 1.0535x over previous
import dataclasses

import jax
import jax.numpy as jnp
from jax import lax
from jax.experimental import pallas as pl
from jax.experimental.pallas import tpu as pltpu
from jax.experimental.pallas import tpu_sc as plsc

N = 10000
E = 320000
D = 128
NC = 2
NS = 16
NW = NC * NS
PER_W = E // NW
CH = 40
NCH = PER_W // CH
ROWS_A = 624
TAIL0 = NS * ROWS_A
TAIL = N - TAIL0


def _sc_segsum(with_counts=False):
    NB = 4 if with_counts else 6
    mesh = plsc.VectorSubcoreMesh(core_axis_name="c", subcore_axis_name="s")
    out_type = [jax.ShapeDtypeStruct((NC, N, D), jnp.float32)]
    scratch = (
        [pltpu.VMEM_SHARED((N, D), jnp.float32)]
        + [pltpu.VMEM((PER_W,), jnp.int32)] * 2
        + [pltpu.VMEM((CH, D), jnp.float32)] * NB
        + [pltpu.SemaphoreType.DMA] * (2 * NB)
    )
    cp = None
    if with_counts:
        out_type.append(jax.ShapeDtypeStruct((NW, N), jnp.float32))
        scratch.append(pltpu.VMEM((N,), jnp.float32))
        cp = pltpu.CompilerParams()
        if "needs_layout_passes" in pltpu.CompilerParams.__dataclass_fields__:
            cp = dataclasses.replace(cp, needs_layout_passes=False)

    def body(x_hbm, src_hbm, dst_hbm, zacc_hbm, *rest):
        if with_counts:
            acc_out, cnt_out, acc_sh, src_v, dst_v = rest[:5]
            bufs_and_sems = rest[5:]
            cnt_v = bufs_and_sems[3 * NB]
        else:
            acc_out, acc_sh, src_v, dst_v = rest[:4]
            bufs_and_sems = rest[4:]
        rows = bufs_and_sems[:NB]
        gsem = bufs_and_sems[NB:2 * NB]
        ssem = bufs_and_sems[2 * NB:3 * NB]
        cid = lax.axis_index("c")
        sid = lax.axis_index("s")
        wid = cid * NS + sid
        row0 = sid * ROWS_A

        pltpu.sync_copy(src_hbm.at[wid], src_v)
        pltpu.sync_copy(dst_hbm.at[wid], dst_v)
        pltpu.sync_copy(zacc_hbm.at[pl.ds(row0, ROWS_A)],
                        acc_sh.at[pl.ds(row0, ROWS_A)])

        @pl.when(sid == NS - 1)
        def _():
            pltpu.sync_copy(zacc_hbm.at[pl.ds(TAIL0, TAIL)],
                            acc_sh.at[pl.ds(TAIL0, TAIL)])

        if with_counts:
            @pl.loop(0, N, step=16)
            def _(j):
                cnt_v[pl.ds(j, 16)] = jnp.zeros((16,), jnp.float32)

        plsc.subcore_barrier()

        def counts(c):
            if with_counts:
                ones16 = jnp.ones((16,), jnp.float32)
                for j in range(CH // 16):
                    plsc.addupdate_scatter(
                        cnt_v, [dst_v[pl.ds(c * CH + j * 16, 16)]], ones16)
                rem = CH % 16
                if rem:
                    mask = lax.iota(jnp.int32, 16) >= (16 - rem)
                    plsc.addupdate_scatter(
                        cnt_v, [dst_v[pl.ds(c * CH + CH - 16, 16)]], ones16,
                        mask=mask)

        def start_gather(c, b):
            pltpu.async_copy(
                x_hbm.at[src_v.at[pl.ds(c * CH, CH)]], rows[b], gsem[b])

        def wait_gather(c, b):
            pltpu.make_async_copy(
                x_hbm.at[src_v.at[pl.ds(c * CH, CH)]], rows[b],
                gsem[b]).wait()

        def start_scatter(c, b):
            pltpu.async_copy(
                rows[b], acc_sh.at[dst_v.at[pl.ds(c * CH, CH)]], ssem[b],
                add=True)

        def wait_scatter(c, b):
            pltpu.make_async_copy(
                rows[b], acc_sh.at[dst_v.at[pl.ds(c * CH, CH)]],
                ssem[b]).wait()

        for c in range(NB - 1):
            start_gather(c, c)

        MAIN = NCH - NCH % NB

        @pl.loop(0, MAIN // NB)
        def _(k):
            for j in range(NB):
                b = j
                c = NB * k + j
                wait_gather(c, b)
                start_scatter(c, b)
                if j == 0:
                    @pl.when(k > 0)
                    def _():
                        wait_scatter(c - 1, (b - 1) % NB)
                else:
                    wait_scatter(c - 1, (b - 1) % NB)

                @pl.when(c + NB - 1 < NCH)
                def _():
                    start_gather(c + NB - 1, (j + NB - 1) % NB)

                counts(c)

        for c in range(MAIN, NCH):
            b = c % NB
            wait_gather(c, b)
            start_scatter(c, b)
            wait_scatter(c - 1, (b - 1) % NB)
            counts(c)
        wait_scatter(NCH - 1, (NCH - 1) % NB)

        plsc.subcore_barrier()

        pltpu.sync_copy(acc_sh.at[pl.ds(row0, ROWS_A)],
                        acc_out.at[cid].at[pl.ds(row0, ROWS_A)])

        @pl.when(sid == NS - 1)
        def _():
            pltpu.sync_copy(acc_sh.at[pl.ds(TAIL0, TAIL)],
                            acc_out.at[cid].at[pl.ds(TAIL0, TAIL)])

        if with_counts:
            pltpu.sync_copy(cnt_v, cnt_out.at[wid])

    return pl.kernel(body, out_type=tuple(out_type), mesh=mesh,
                     scratch_types=scratch, compiler_params=cp)


def _sc_hist():
    mesh = plsc.VectorSubcoreMesh(core_axis_name="c", subcore_axis_name="s")
    out_type = jax.ShapeDtypeStruct((NW, N), jnp.float32)
    scratch = [
        pltpu.VMEM((PER_W,), jnp.int32),
        pltpu.VMEM((N,), jnp.float32),
    ]
    cp = pltpu.CompilerParams()
    if "needs_layout_passes" in pltpu.CompilerParams.__dataclass_fields__:
        cp = dataclasses.replace(cp, needs_layout_passes=False)

    def body(dst_hbm, cnt_out, dst_v, cnt_v):
        cid = lax.axis_index("c")
        sid = lax.axis_index("s")
        wid = cid * NS + sid

        pltpu.sync_copy(dst_hbm.at[wid], dst_v)

        @pl.loop(0, N, step=16)
        def _(j):
            cnt_v[pl.ds(j, 16)] = jnp.zeros((16,), jnp.float32)

        ones16 = jnp.ones((16,), jnp.float32)

        @pl.loop(0, PER_W, step=16)
        def _(j):
            plsc.addupdate_scatter(cnt_v, [dst_v[pl.ds(j, 16)]], ones16)

        pltpu.sync_copy(cnt_v, cnt_out.at[wid])

    return pl.kernel(body, out_type=out_type, mesh=mesh,
                     scratch_types=scratch, compiler_params=cp)


def _tc_layer(relu):
    B = 1000

    def body(acc_ref, cnt_ref, x_ref, wl_ref, b_ref, wr_ref, o_ref):
        s = acc_ref[0] + acc_ref[1]
        cnt = jnp.sum(cnt_ref[...], axis=1)
        mean = s * (1.0 / jnp.maximum(cnt, 1.0))[:, None]
        dn = (((1,), (1,)), ((), ()))
        r = (lax.dot_general(mean, wl_ref[...], dn,
                             preferred_element_type=jnp.float32)
             + lax.dot_general(x_ref[...], wr_ref[...], dn,
                               preferred_element_type=jnp.float32)
             + b_ref[...])
        o_ref[...] = jnp.maximum(r, 0.0) if relu else r

    return pl.pallas_call(
        body,
        grid=(N // B,),
        in_specs=[
            pl.BlockSpec((NC, B, D), lambda i: (0, i, 0)),
            pl.BlockSpec((B, NW), lambda i: (i, 0)),
            pl.BlockSpec((B, D), lambda i: (i, 0)),
            pl.BlockSpec((D, D), lambda i: (0, 0)),
            pl.BlockSpec((1, D), lambda i: (0, 0)),
            pl.BlockSpec((D, D), lambda i: (0, 0)),
        ],
        out_specs=pl.BlockSpec((B, D), lambda i: (i, 0)),
        out_shape=jax.ShapeDtypeStruct((N, D), jnp.float32),
    )


@jax.jit
def kernel(x, edge_index, W1_l, b1_l, W1_r, W2_l, b2_l, W2_r):
    ei = edge_index.astype(jnp.int32)
    src = ei[0].reshape(NW, PER_W)
    dst2 = ei[1].reshape(NW, PER_W)
    zacc = jnp.zeros((N, D), jnp.float32)
    b1 = b1_l.reshape(1, D)
    b2 = b2_l.reshape(1, D)

    cntp = _sc_hist()(dst2).T
    (acc1,) = _sc_segsum(False)(x, src, dst2, zacc)
    h = _tc_layer(True)(acc1, cntp, x, W1_l, b1, W1_r)
    (acc2,) = _sc_segsum(False)(h, src, dst2, zacc)
    out = _tc_layer(False)(acc2, cntp, h, W2_l, b2, W2_r)
    return out


# speedup vs baseline: 1.0555x; 1.0019x over previous
import dataclasses

import jax
import jax.numpy as jnp
from jax import lax
from jax.experimental import pallas as pl
from jax.experimental.pallas import tpu as pltpu
from jax.experimental.pallas import tpu_sc as plsc

N = 10000
E = 320000
D = 128
NC = 2
NS = 16
NW = NC * NS
PER_W = E // NW
CH = 40
NCH = PER_W // CH
ROWS_A = 624
TAIL0 = NS * ROWS_A
TAIL = N - TAIL0


def _sc_segsum(with_counts=False):
    NB = 4 if with_counts else 6
    mesh = plsc.VectorSubcoreMesh(core_axis_name="c", subcore_axis_name="s")
    out_type = [jax.ShapeDtypeStruct((NC, N, D), jnp.float32)]
    scratch = (
        [pltpu.VMEM_SHARED((N, D), jnp.float32)]
        + [pltpu.VMEM((PER_W,), jnp.int32)] * 2
        + [pltpu.VMEM((CH, D), jnp.float32)] * NB
        + [pltpu.SemaphoreType.DMA] * (2 * NB)
    )
    cp = None
    if with_counts:
        out_type.append(jax.ShapeDtypeStruct((NW, N), jnp.float32))
        scratch.append(pltpu.VMEM((N,), jnp.float32))
        cp = pltpu.CompilerParams()
        if "needs_layout_passes" in pltpu.CompilerParams.__dataclass_fields__:
            cp = dataclasses.replace(cp, needs_layout_passes=False)

    def body(x_hbm, src_hbm, dst_hbm, zacc_hbm, *rest):
        if with_counts:
            acc_out, cnt_out, acc_sh, src_v, dst_v = rest[:5]
            bufs_and_sems = rest[5:]
            cnt_v = bufs_and_sems[3 * NB]
        else:
            acc_out, acc_sh, src_v, dst_v = rest[:4]
            bufs_and_sems = rest[4:]
        rows = bufs_and_sems[:NB]
        gsem = bufs_and_sems[NB:2 * NB]
        ssem = bufs_and_sems[2 * NB:3 * NB]
        cid = lax.axis_index("c")
        sid = lax.axis_index("s")
        wid = cid * NS + sid
        row0 = sid * ROWS_A

        pltpu.sync_copy(src_hbm.at[wid], src_v)
        pltpu.sync_copy(dst_hbm.at[wid], dst_v)
        pltpu.sync_copy(zacc_hbm.at[pl.ds(row0, ROWS_A)],
                        acc_sh.at[pl.ds(row0, ROWS_A)])

        @pl.when(sid == NS - 1)
        def _():
            pltpu.sync_copy(zacc_hbm.at[pl.ds(TAIL0, TAIL)],
                            acc_sh.at[pl.ds(TAIL0, TAIL)])

        if with_counts:
            @pl.loop(0, N, step=16)
            def _(j):
                cnt_v[pl.ds(j, 16)] = jnp.zeros((16,), jnp.float32)

        plsc.subcore_barrier()

        def counts(c):
            if with_counts:
                ones16 = jnp.ones((16,), jnp.float32)
                for j in range(CH // 16):
                    plsc.addupdate_scatter(
                        cnt_v, [dst_v[pl.ds(c * CH + j * 16, 16)]], ones16)
                rem = CH % 16
                if rem:
                    mask = lax.iota(jnp.int32, 16) >= (16 - rem)
                    plsc.addupdate_scatter(
                        cnt_v, [dst_v[pl.ds(c * CH + CH - 16, 16)]], ones16,
                        mask=mask)

        def start_gather(c, b):
            pltpu.async_copy(
                x_hbm.at[src_v.at[pl.ds(c * CH, CH)]], rows[b], gsem[b])

        def wait_gather(c, b):
            pltpu.make_async_copy(
                x_hbm.at[src_v.at[pl.ds(c * CH, CH)]], rows[b],
                gsem[b]).wait()

        def start_scatter(c, b):
            pltpu.async_copy(
                rows[b], acc_sh.at[dst_v.at[pl.ds(c * CH, CH)]], ssem[b],
                add=True)

        def wait_scatter(c, b):
            pltpu.make_async_copy(
                rows[b], acc_sh.at[dst_v.at[pl.ds(c * CH, CH)]],
                ssem[b]).wait()

        for c in range(NB - 1):
            start_gather(c, c)

        MAIN = NCH - NCH % NB

        @pl.loop(0, MAIN // NB)
        def _(k):
            for j in range(NB):
                b = j
                c = NB * k + j
                wait_gather(c, b)
                start_scatter(c, b)
                if j == 0:
                    @pl.when(k > 0)
                    def _():
                        wait_scatter(c - 1, (b - 1) % NB)
                else:
                    wait_scatter(c - 1, (b - 1) % NB)

                @pl.when(c + NB - 1 < NCH)
                def _():
                    start_gather(c + NB - 1, (j + NB - 1) % NB)

                counts(c)

        for c in range(MAIN, NCH):
            b = c % NB
            wait_gather(c, b)
            start_scatter(c, b)
            wait_scatter(c - 1, (b - 1) % NB)
            counts(c)
        wait_scatter(NCH - 1, (NCH - 1) % NB)

        plsc.subcore_barrier()

        pltpu.sync_copy(acc_sh.at[pl.ds(row0, ROWS_A)],
                        acc_out.at[cid].at[pl.ds(row0, ROWS_A)])

        @pl.when(sid == NS - 1)
        def _():
            pltpu.sync_copy(acc_sh.at[pl.ds(TAIL0, TAIL)],
                            acc_out.at[cid].at[pl.ds(TAIL0, TAIL)])

        if with_counts:
            pltpu.sync_copy(cnt_v, cnt_out.at[wid])

    return pl.kernel(body, out_type=tuple(out_type), mesh=mesh,
                     scratch_types=scratch, compiler_params=cp)


def _sc_hist():
    mesh = plsc.VectorSubcoreMesh(core_axis_name="c", subcore_axis_name="s")
    out_type = jax.ShapeDtypeStruct((NW, N), jnp.float32)
    scratch = [
        pltpu.VMEM((PER_W,), jnp.int32),
        pltpu.VMEM((N,), jnp.float32),
    ]
    cp = pltpu.CompilerParams()
    if "needs_layout_passes" in pltpu.CompilerParams.__dataclass_fields__:
        cp = dataclasses.replace(cp, needs_layout_passes=False)

    def body(dst_hbm, cnt_out, dst_v, cnt_v):
        cid = lax.axis_index("c")
        sid = lax.axis_index("s")
        wid = cid * NS + sid

        pltpu.sync_copy(dst_hbm.at[wid], dst_v)

        @pl.loop(0, N, step=16)
        def _(j):
            cnt_v[pl.ds(j, 16)] = jnp.zeros((16,), jnp.float32)

        ones16 = jnp.ones((16,), jnp.float32)

        @pl.loop(0, PER_W, step=16)
        def _(j):
            plsc.addupdate_scatter(cnt_v, [dst_v[pl.ds(j, 16)]], ones16)

        pltpu.sync_copy(cnt_v, cnt_out.at[wid])

    return pl.kernel(body, out_type=out_type, mesh=mesh,
                     scratch_types=scratch, compiler_params=cp)


B = 1000


def _tc_right():
    def body(x_ref, wr_ref, b_ref, o_ref):
        dn = (((1,), (1,)), ((), ()))
        o_ref[...] = lax.dot_general(
            x_ref[...], wr_ref[...], dn,
            preferred_element_type=jnp.float32) + b_ref[...]

    return pl.pallas_call(
        body,
        grid=(N // B,),
        in_specs=[
            pl.BlockSpec((B, D), lambda i: (i, 0)),
            pl.BlockSpec((D, D), lambda i: (0, 0)),
            pl.BlockSpec((1, D), lambda i: (0, 0)),
        ],
        out_specs=pl.BlockSpec((B, D), lambda i: (i, 0)),
        out_shape=jax.ShapeDtypeStruct((N, D), jnp.float32),
    )


def _tc_left(relu):
    def body(acc_ref, cnt_ref, xr_ref, wl_ref, o_ref):
        s = acc_ref[0] + acc_ref[1]
        cnt = jnp.sum(cnt_ref[...], axis=1)
        mean = s * (1.0 / jnp.maximum(cnt, 1.0))[:, None]
        dn = (((1,), (1,)), ((), ()))
        r = lax.dot_general(mean, wl_ref[...], dn,
                            preferred_element_type=jnp.float32) + xr_ref[...]
        o_ref[...] = jnp.maximum(r, 0.0) if relu else r

    return pl.pallas_call(
        body,
        grid=(N // B,),
        in_specs=[
            pl.BlockSpec((NC, B, D), lambda i: (0, i, 0)),
            pl.BlockSpec((B, NW), lambda i: (i, 0)),
            pl.BlockSpec((B, D), lambda i: (i, 0)),
            pl.BlockSpec((D, D), lambda i: (0, 0)),
        ],
        out_specs=pl.BlockSpec((B, D), lambda i: (i, 0)),
        out_shape=jax.ShapeDtypeStruct((N, D), jnp.float32),
    )


@jax.jit
def kernel(x, edge_index, W1_l, b1_l, W1_r, W2_l, b2_l, W2_r):
    ei = edge_index.astype(jnp.int32)
    src = ei[0].reshape(NW, PER_W)
    dst2 = ei[1].reshape(NW, PER_W)
    zacc = jnp.zeros((N, D), jnp.float32)
    b1 = b1_l.reshape(1, D)
    b2 = b2_l.reshape(1, D)

    cntp = _sc_hist()(dst2).T
    (acc1,) = _sc_segsum(False)(x, src, dst2, zacc)
    xr1 = _tc_right()(x, W1_r, b1)
    h = _tc_left(True)(acc1, cntp, xr1, W1_l)
    (acc2,) = _sc_segsum(False)(h, src, dst2, zacc)
    xr2 = _tc_right()(h, W2_r, b2)
    out = _tc_left(False)(acc2, cntp, xr2, W2_l)
    return out
